# baseline (speedup 1.0000x reference)
import jax
import jax.numpy as jnp
from jax.experimental import pallas as pl
from jax.experimental.pallas import tpu as pltpu

NFEAT = 128
NHID = 32
ALPHA = 0.2
N_NODES = 10000
N_CLUSTERS = 10

BR = 200
NI = N_NODES // BR
NBUF = 4


def _soft_assign(o, mu):
    cols = []
    for c in range(N_CLUSTERS):
        d = o - mu[c:c + 1, :]
        cols.append(jnp.sum(d * d, axis=1, keepdims=True))
    dist2 = jnp.concatenate(cols, axis=1)
    qv = 1.0 / (1.0 + dist2 / ALPHA + 1e-8)
    p = jnp.exp((ALPHA + 1.0) * jnp.log(qv))
    return p / jnp.sum(p, axis=1, keepdims=True)


def _adj_copy(adj_hbm, adj_buf, adj_sem, block, slot):
    return pltpu.make_async_copy(
        adj_hbm.at[pl.ds(block * BR, BR), :], adj_buf.at[slot], adj_sem.at[slot])


def _out_copies(out_buf, q_buf, out_hbm, q_hbm, out_sem, q_sem, block, slot):
    rows = pl.ds(block * BR, BR)
    return (
        pltpu.make_async_copy(out_buf.at[slot], out_hbm.at[rows, :], out_sem.at[slot]),
        pltpu.make_async_copy(q_buf.at[slot], q_hbm.at[rows, :], q_sem.at[slot]),
    )


def _gcdec_kernel(x_hbm, adj_hbm, w_ref, b_ref, mu_ref, out_hbm, q_hbm,
                  x_vmem, sup_ref, adj_buf, out_buf, q_buf,
                  x_sem, adj_sem, out_sem, q_sem):
    cp_x = pltpu.make_async_copy(x_hbm, x_vmem, x_sem)
    cp_x.start()
    for j in range(NBUF):
        _adj_copy(adj_hbm, adj_buf, adj_sem, j, j).start()
    cp_x.wait()
    sup_ref[...] = jnp.dot(x_vmem[...], w_ref[...],
                           preferred_element_type=jnp.float32)

    def body(i, carry):
        slot = jax.lax.rem(i, NBUF)
        oslot = jax.lax.rem(i, 2)
        _adj_copy(adj_hbm, adj_buf, adj_sem, i, slot).wait()
        o = jnp.dot(adj_buf[slot], sup_ref[...],
                    preferred_element_type=jnp.float32) + b_ref[...]
        q = _soft_assign(o, mu_ref[...])

        @pl.when(i >= 2)
        def _():
            co, cq = _out_copies(out_buf, q_buf, out_hbm, q_hbm,
                                 out_sem, q_sem, i - 2, oslot)
            co.wait()
            cq.wait()

        out_buf[oslot] = o
        q_buf[oslot] = q
        co, cq = _out_copies(out_buf, q_buf, out_hbm, q_hbm,
                             out_sem, q_sem, i, oslot)
        co.start()
        cq.start()

        @pl.when(i + NBUF < NI)
        def _():
            _adj_copy(adj_hbm, adj_buf, adj_sem, i + NBUF, slot).start()

        return carry

    jax.lax.fori_loop(0, NI, body, 0)
    for i in (NI - 2, NI - 1):
        co, cq = _out_copies(out_buf, q_buf, out_hbm, q_hbm,
                             out_sem, q_sem, i, i % 2)
        co.wait()
        cq.wait()


@jax.jit
def kernel(x, adj, W, b, mu):
    b2 = b.reshape(1, NHID)
    out, q = pl.pallas_call(
        _gcdec_kernel,
        in_specs=[
            pl.BlockSpec(memory_space=pl.ANY),
            pl.BlockSpec(memory_space=pl.ANY),
            pl.BlockSpec((NFEAT, NHID), lambda: (0, 0)),
            pl.BlockSpec((1, NHID), lambda: (0, 0)),
            pl.BlockSpec((N_CLUSTERS, NHID), lambda: (0, 0)),
        ],
        out_specs=[
            pl.BlockSpec(memory_space=pl.ANY),
            pl.BlockSpec(memory_space=pl.ANY),
        ],
        out_shape=[
            jax.ShapeDtypeStruct((N_NODES, NHID), jnp.float32),
            jax.ShapeDtypeStruct((N_NODES, N_CLUSTERS), jnp.float32),
        ],
        scratch_shapes=[
            pltpu.VMEM((N_NODES, NFEAT), jnp.float32),
            pltpu.VMEM((N_NODES, NHID), jnp.float32),
            pltpu.VMEM((NBUF, BR, N_NODES), jnp.float32),
            pltpu.VMEM((2, BR, NHID), jnp.float32),
            pltpu.VMEM((2, BR, N_CLUSTERS), jnp.float32),
            pltpu.SemaphoreType.DMA,
            pltpu.SemaphoreType.DMA((NBUF,)),
            pltpu.SemaphoreType.DMA((2,)),
            pltpu.SemaphoreType.DMA((2,)),
        ],
    )(x, adj, W, b2, mu)
    return (out, q)
